# baseline (speedup 1.0000x reference)
import jax
import jax.numpy as jnp
from jax import lax
from jax.experimental import pallas as pl
from jax.experimental.pallas import tpu as pltpu
from jax.experimental.pallas import tpu_sc as plsc

B = 4096
L = 200
D = 32

_NC = 2
_NS = 16
_NW = _NC * _NS
_BB = B // _NW
_T = D // 8


def _transpose_to_tiles(src, dst, iota16):
    def per_d(d, carry):
        i_t = jnp.full((16,), d // 8, jnp.int32)
        i_s = jnp.full((16,), d % 8, jnp.int32)
        i_d = jnp.full((16,), d, jnp.int32)
        for g in range(8):
            c = iota16 + g * 16
            v = plsc.load_gather(src, [c, i_d])
            plsc.store_scatter(dst, [i_t, i_s, c], v)
        return carry
    lax.fori_loop(0, D, per_d, 0)


def _din_body(uid_hbm, iid_hbm, sqT_hbm, ut_hbm, it_hbm,
              u5_out, i5_out, s5_out,
              uidx_v, urows_v, iidx_v, irows_v,
              idx0, idx1, rows0, rows1, tr0, tr1, utr_v,
              usem, isem, g0, g1, t0, t1):
    c = lax.axis_index("c")
    s = lax.axis_index("s")
    wid = s * _NC + c
    b0 = wid * _BB
    iota16 = lax.iota(jnp.int32, 16)

    pltpu.sync_copy(uid_hbm.at[pl.ds(b0, _BB)], uidx_v)
    ucopy = pltpu.make_async_copy(ut_hbm.at[uidx_v], urows_v, usem)
    ucopy.start()
    pltpu.sync_copy(iid_hbm.at[pl.ds(b0, _BB)], iidx_v)
    icopy = pltpu.make_async_copy(it_hbm.at[iidx_v], irows_v, isem)
    icopy.start()

    idx = (idx0, idx1)
    rows = (rows0, rows1)
    tr = (tr0, tr1)
    gsem = (g0, g1)
    tsem = (t0, t1)

    def load_and_gather(l, b):
        pltpu.sync_copy(sqT_hbm.at[l, pl.ds(b0, _BB)], idx[b])
        pltpu.make_async_copy(it_hbm.at[idx[b]], rows[b], gsem[b]).start()

    def gwait(b):
        pltpu.make_async_copy(it_hbm.at[idx[b]], rows[b], gsem[b]).wait()

    def writeback(l, b):
        return pltpu.make_async_copy(tr[b], s5_out.at[l, :, wid], tsem[b])

    load_and_gather(0, 0)
    load_and_gather(1, 1)

    for b in range(2):
        gwait(b)
        _transpose_to_tiles(rows[b], tr[b], iota16)
        load_and_gather(b + 2, b)
        writeback(b, b).start()

    def steady(ll, carry):
        for b in range(2):
            l = 2 * ll + b
            gwait(b)
            writeback(l - 2, b).wait()
            _transpose_to_tiles(rows[b], tr[b], iota16)
            load_and_gather(l + 2, b)
            writeback(l, b).start()
        return carry

    lax.fori_loop(1, (L - 2) // 2, steady, 0)

    for b in range(2):
        l = L - 2 + b
        gwait(b)
        writeback(l - 2, b).wait()
        _transpose_to_tiles(rows[b], tr[b], iota16)
        writeback(l, b).start()
    writeback(L - 2, 0).wait()
    writeback(L - 1, 1).wait()

    ucopy.wait()
    _transpose_to_tiles(urows_v, utr_v, iota16)
    pltpu.sync_copy(utr_v, u5_out.at[:, wid])
    icopy.wait()
    _transpose_to_tiles(irows_v, utr_v, iota16)
    pltpu.sync_copy(utr_v, i5_out.at[:, wid])


@jax.jit
def _din_sc(uid_idx, iid_idx, seq_idx_T, user_table, item_table):
    mesh = plsc.VectorSubcoreMesh(core_axis_name="c", subcore_axis_name="s")
    f = pl.kernel(
        _din_body,
        out_type=(
            jax.ShapeDtypeStruct((_T, _NW, 8, 128), jnp.float32),
            jax.ShapeDtypeStruct((_T, _NW, 8, 128), jnp.float32),
            jax.ShapeDtypeStruct((L, _T, _NW, 8, 128), jnp.float32),
        ),
        mesh=mesh,
        compiler_params=pltpu.CompilerParams(
            use_tc_tiling_on_sc=False, needs_layout_passes=False),
        scratch_types=[
            pltpu.VMEM((_BB,), jnp.int32),
            pltpu.VMEM((_BB, D), jnp.float32),
            pltpu.VMEM((_BB,), jnp.int32),
            pltpu.VMEM((_BB, D), jnp.float32),
            pltpu.VMEM((_BB,), jnp.int32),
            pltpu.VMEM((_BB,), jnp.int32),
            pltpu.VMEM((_BB, D), jnp.float32),
            pltpu.VMEM((_BB, D), jnp.float32),
            pltpu.VMEM((_T, 8, 128), jnp.float32),
            pltpu.VMEM((_T, 8, 128), jnp.float32),
            pltpu.VMEM((_T, 8, 128), jnp.float32),
            pltpu.SemaphoreType.DMA,
            pltpu.SemaphoreType.DMA,
            pltpu.SemaphoreType.DMA,
            pltpu.SemaphoreType.DMA,
            pltpu.SemaphoreType.DMA,
            pltpu.SemaphoreType.DMA,
        ],
    )
    return f(uid_idx, iid_idx, seq_idx_T, user_table, item_table)


def kernel(uid_idx, iid_idx, seq_idx, mask, dense, user_table, item_table):
    del mask, dense
    u5, i5, s5 = _din_sc(
        uid_idx.astype(jnp.int32), iid_idx.astype(jnp.int32),
        seq_idx.astype(jnp.int32).T, user_table, item_table)
    user_embed = u5.transpose(1, 3, 0, 2).reshape(B, D)
    item_embed = i5.transpose(1, 3, 0, 2).reshape(B, D)
    seq_embed = s5.transpose(2, 4, 0, 1, 3).reshape(B, L, D)
    return (user_embed, item_embed, seq_embed)


# speedup vs baseline: 1.1678x; 1.1678x over previous
import jax
import jax.numpy as jnp
from jax import lax
from jax.experimental import pallas as pl
from jax.experimental.pallas import tpu as pltpu
from jax.experimental.pallas import tpu_sc as plsc

B = 4096
L = 200
D = 32

_NC = 2
_NS = 16
_NW = _NC * _NS

_PER_W_B = B // _NW
_SEQ_TOTAL = B * L
_PER_W_SEQ = _SEQ_TOTAL // _NW
_CH = 1600
_NCH = _PER_W_SEQ // _CH


def _din_body(uid_hbm, iid_hbm, seq_hbm, ut_hbm, it_hbm,
              user_out, item_out, seq_out,
              uidx_v, urows_v, iidx_v, irows_v,
              idx0, idx1, rows0, rows1,
              usem, isem, g0, g1, o0, o1):
    c = lax.axis_index("c")
    s = lax.axis_index("s")
    wid = s * _NC + c
    ub = wid * _PER_W_B
    sbase = wid * _PER_W_SEQ

    pltpu.sync_copy(uid_hbm.at[pl.ds(ub, _PER_W_B)], uidx_v)
    ucopy = pltpu.make_async_copy(ut_hbm.at[uidx_v], urows_v, usem)
    ucopy.start()
    pltpu.sync_copy(iid_hbm.at[pl.ds(ub, _PER_W_B)], iidx_v)
    icopy = pltpu.make_async_copy(it_hbm.at[iidx_v], irows_v, isem)
    icopy.start()

    idx = (idx0, idx1)
    rows = (rows0, rows1)
    gsem = (g0, g1)
    osem = (o0, o1)

    def load_and_gather(j, b):
        pltpu.sync_copy(seq_hbm.at[pl.ds(sbase + j * _CH, _CH)], idx[b])
        pltpu.make_async_copy(it_hbm.at[idx[b]], rows[b], gsem[b]).start()

    def writeback(j, b):
        return pltpu.make_async_copy(
            rows[b], seq_out.at[pl.ds(sbase + j * _CH, _CH)], osem[b])

    load_and_gather(0, 0)
    load_and_gather(1, 1)

    def outer(jj, carry):
        for b in range(2):
            j = 2 * jj + b
            pltpu.make_async_copy(it_hbm.at[idx[b]], rows[b], gsem[b]).wait()
            wb = writeback(j, b)
            wb.start()
            wb.wait()
            load_and_gather(j + 2, b)
        return carry

    lax.fori_loop(0, (_NCH - 2) // 2, outer, 0)

    pltpu.make_async_copy(it_hbm.at[idx[0]], rows[0], gsem[0]).wait()
    writeback(_NCH - 2, 0).start()
    pltpu.make_async_copy(it_hbm.at[idx[1]], rows[1], gsem[1]).wait()
    writeback(_NCH - 1, 1).start()
    writeback(_NCH - 2, 0).wait()
    writeback(_NCH - 1, 1).wait()

    ucopy.wait()
    pltpu.sync_copy(urows_v, user_out.at[pl.ds(ub, _PER_W_B)])
    icopy.wait()
    pltpu.sync_copy(irows_v, item_out.at[pl.ds(ub, _PER_W_B)])


@jax.jit
def _din_sc(uid_idx, iid_idx, seq_flat, user_table, item_table):
    mesh = plsc.VectorSubcoreMesh(core_axis_name="c", subcore_axis_name="s")
    f = pl.kernel(
        _din_body,
        out_type=(
            jax.ShapeDtypeStruct((B, D), jnp.float32),
            jax.ShapeDtypeStruct((B, D), jnp.float32),
            jax.ShapeDtypeStruct((_SEQ_TOTAL, D), jnp.float32),
        ),
        mesh=mesh,
        compiler_params=pltpu.CompilerParams(use_tc_tiling_on_sc=False),
        scratch_types=[
            pltpu.VMEM((_PER_W_B,), jnp.int32),
            pltpu.VMEM((_PER_W_B, D), jnp.float32),
            pltpu.VMEM((_PER_W_B,), jnp.int32),
            pltpu.VMEM((_PER_W_B, D), jnp.float32),
            pltpu.VMEM((_CH,), jnp.int32),
            pltpu.VMEM((_CH,), jnp.int32),
            pltpu.VMEM((_CH, D), jnp.float32),
            pltpu.VMEM((_CH, D), jnp.float32),
            pltpu.SemaphoreType.DMA,
            pltpu.SemaphoreType.DMA,
            pltpu.SemaphoreType.DMA,
            pltpu.SemaphoreType.DMA,
            pltpu.SemaphoreType.DMA,
            pltpu.SemaphoreType.DMA,
        ],
    )
    return f(uid_idx, iid_idx, seq_flat, user_table, item_table)


def kernel(uid_idx, iid_idx, seq_idx, mask, dense, user_table, item_table):
    del mask, dense
    seq_flat = seq_idx.reshape(_SEQ_TOTAL).astype(jnp.int32)
    user_embed, item_embed, seq_embed = _din_sc(
        uid_idx.astype(jnp.int32), iid_idx.astype(jnp.int32), seq_flat,
        user_table, item_table)
    return (user_embed, item_embed, seq_embed.reshape(B, L, D))
